# baseline (speedup 1.0000x reference)
import functools

import jax
import jax.numpy as jnp
from jax import lax
from jax.experimental import pallas as pl
from jax.experimental.pallas import tpu as pltpu
from jax.experimental.pallas import tpu_sc as plsc

N_TOTAL = 8388608
NUM_CLASSES = 2
BETA = 0.99

NC = 2
NS = 16
NW = NC * NS
LANES = 16
UNROLL = 4
CHUNK = 65536
NBUF = 1

N_SC = 1 * NW * CHUNK
PER_W = N_SC // NW
NCH = PER_W // CHUNK

N_TC = N_TOTAL - N_SC
TC_COLS = 128
TC_ROWS = N_TC // TC_COLS
TC_BLOCK_ROWS = 16384
TC_GRID = TC_ROWS // TC_BLOCK_ROWS
TC_BLOCK_OFF = N_SC // (TC_COLS * TC_BLOCK_ROWS)


def _sc_partial_sums_body(y_hbm, part_hbm, buf0, acc_v, sem0):
    c = lax.axis_index("c")
    s = lax.axis_index("s")
    wid = s * NC + c
    base = wid * PER_W

    bufs = (buf0,)
    sems = (sem0,)

    for b in range(min(NBUF, NCH)):
        pltpu.make_async_copy(
            y_hbm.at[pl.ds(base + b * CHUNK, CHUNK)], bufs[b], sems[b]
        ).start()

    accs = tuple(jnp.zeros((LANES,), jnp.int32) for _ in range(UNROLL))

    for i in range(NCH):
        slot = i % NBUF
        pltpu.make_async_copy(
            y_hbm.at[pl.ds(base + i * CHUNK, CHUNK)], bufs[slot], sems[slot]
        ).wait()
        bref = bufs[slot]

        def body(j, accs, bref=bref):
            off = j * (LANES * UNROLL)
            return tuple(
                a + bref[pl.ds(off + k * LANES, LANES)]
                for k, a in enumerate(accs)
            )

        accs = plsc.parallel_loop(
            0, CHUNK // (LANES * UNROLL), 1, unroll=4, carry=accs
        )(body)

        nxt = i + NBUF
        if nxt < NCH:
            pltpu.make_async_copy(
                y_hbm.at[pl.ds(base + nxt * CHUNK, CHUNK)], bufs[slot], sems[slot]
            ).start()

    acc = accs[0]
    for a in accs[1:]:
        acc = acc + a
    acc_v[...] = acc
    pltpu.sync_copy(acc_v, part_hbm.at[pl.ds(wid * LANES, LANES)])


@functools.cache
def _sc_partial_sums():
    return pl.kernel(
        _sc_partial_sums_body,
        out_type=jax.ShapeDtypeStruct((NW * LANES,), jnp.int32),
        mesh=plsc.VectorSubcoreMesh(
            core_axis_name="c", subcore_axis_name="s", num_cores=NC, num_subcores=NS
        ),
        scratch_types=[
            pltpu.VMEM((CHUNK,), jnp.int32),
            pltpu.VMEM((LANES,), jnp.int32),
            pltpu.SemaphoreType.DMA,
        ],
    )


def _tc_reduce_body(y_ref, acc_ref):
    i = pl.program_id(0)

    @pl.when(i == 0)
    def _():
        acc_ref[...] = jnp.zeros_like(acc_ref)

    b = y_ref[...].reshape(4, TC_BLOCK_ROWS // 32, 8, TC_COLS)
    acc_ref[...] += (
        (jnp.sum(b[0], axis=0) + jnp.sum(b[1], axis=0))
        + (jnp.sum(b[2], axis=0) + jnp.sum(b[3], axis=0))
    )


_tc_reduce = pl.pallas_call(
    _tc_reduce_body,
    grid=(TC_GRID,),
    in_specs=[pl.BlockSpec((TC_BLOCK_ROWS, TC_COLS), lambda i: (TC_BLOCK_OFF + i, 0))],
    out_specs=pl.BlockSpec(memory_space=pltpu.VMEM),
    out_shape=jax.ShapeDtypeStruct((8, TC_COLS), jnp.int32),
)


def _tc_finalize_body(part_ref, tcsum_ref, counts_ref, nc_ref, w_ref):
    total_pos = (jnp.sum(part_ref[...]) + jnp.sum(tcsum_ref[...])).astype(jnp.float32)
    neg = jnp.float32(N_TOTAL) - total_pos
    c0 = counts_ref[0]
    c1 = counts_ref[1]
    n0 = BETA * c0 + (1.0 - BETA) * neg
    n1 = BETA * c1 + (1.0 - BETA) * total_pos
    s0 = n0 + 1.0
    s1 = n1 + 1.0
    tot = s0 + s1
    w0 = tot / (NUM_CLASSES * s0)
    w1 = tot / (NUM_CLASSES * s1)
    wm = (w0 + w1) * 0.5
    w0n = w0 / (wm + 1e-8)
    w1n = w1 / (wm + 1e-8)
    nc_ref[0] = n0
    nc_ref[1] = n1
    w_ref[0] = w0n
    w_ref[1] = w1n


_tc_finalize = pl.pallas_call(
    _tc_finalize_body,
    out_shape=(
        jax.ShapeDtypeStruct((NUM_CLASSES,), jnp.float32),
        jax.ShapeDtypeStruct((NUM_CLASSES,), jnp.float32),
    ),
    in_specs=[
        pl.BlockSpec(memory_space=pltpu.VMEM),
        pl.BlockSpec(memory_space=pltpu.VMEM),
        pl.BlockSpec(memory_space=pltpu.SMEM),
    ],
    out_specs=(
        pl.BlockSpec(memory_space=pltpu.SMEM),
        pl.BlockSpec(memory_space=pltpu.SMEM),
    ),
)


@jax.jit
def kernel(y, counts):
    y = y.astype(jnp.int32)
    part_sc = _sc_partial_sums()(y)
    tc_sum = _tc_reduce(y.reshape(N_TOTAL // TC_COLS, TC_COLS))
    new_counts, weights = _tc_finalize(part_sc.reshape(4, 128), tc_sum, counts)
    return new_counts, weights


# speedup vs baseline: 1.0044x; 1.0044x over previous
import functools

import jax
import jax.numpy as jnp
from jax import lax
from jax.experimental import pallas as pl
from jax.experimental.pallas import tpu as pltpu
from jax.experimental.pallas import tpu_sc as plsc

N_TOTAL = 8388608
NUM_CLASSES = 2
BETA = 0.99

NC = 2
NS = 16
NW = NC * NS
LANES = 16
UNROLL = 4
CHUNK = 32768
NBUF = 2

N_SC = 2 * NW * CHUNK
PER_W = N_SC // NW
NCH = PER_W // CHUNK

N_TC = N_TOTAL - N_SC
TC_COLS = 128
TC_ROWS = N_TC // TC_COLS
TC_BLOCK_ROWS = 16384
TC_GRID = TC_ROWS // TC_BLOCK_ROWS
TC_BLOCK_OFF = N_SC // (TC_COLS * TC_BLOCK_ROWS)


def _sc_partial_sums_body(y_hbm, part_hbm, buf0, buf1, acc_v, sem0, sem1):
    c = lax.axis_index("c")
    s = lax.axis_index("s")
    wid = s * NC + c
    base = wid * PER_W

    bufs = (buf0, buf1)
    sems = (sem0, sem1)

    for b in range(min(NBUF, NCH)):
        pltpu.make_async_copy(
            y_hbm.at[pl.ds(base + b * CHUNK, CHUNK)], bufs[b], sems[b]
        ).start()

    accs = tuple(jnp.zeros((LANES,), jnp.int32) for _ in range(UNROLL))

    for i in range(NCH):
        slot = i % NBUF
        pltpu.make_async_copy(
            y_hbm.at[pl.ds(base + i * CHUNK, CHUNK)], bufs[slot], sems[slot]
        ).wait()
        bref = bufs[slot]

        def body(j, accs, bref=bref):
            off = j * (LANES * UNROLL)
            return tuple(
                a + bref[pl.ds(off + k * LANES, LANES)]
                for k, a in enumerate(accs)
            )

        accs = plsc.parallel_loop(
            0, CHUNK // (LANES * UNROLL), 1, unroll=4, carry=accs
        )(body)

        nxt = i + NBUF
        if nxt < NCH:
            pltpu.make_async_copy(
                y_hbm.at[pl.ds(base + nxt * CHUNK, CHUNK)], bufs[slot], sems[slot]
            ).start()

    acc = accs[0]
    for a in accs[1:]:
        acc = acc + a
    acc_v[...] = acc
    pltpu.sync_copy(acc_v, part_hbm.at[pl.ds(wid * LANES, LANES)])


@functools.cache
def _sc_partial_sums():
    return pl.kernel(
        _sc_partial_sums_body,
        out_type=jax.ShapeDtypeStruct((NW * LANES,), jnp.int32),
        mesh=plsc.VectorSubcoreMesh(
            core_axis_name="c", subcore_axis_name="s", num_cores=NC, num_subcores=NS
        ),
        scratch_types=[
            pltpu.VMEM((CHUNK,), jnp.int32),
            pltpu.VMEM((CHUNK,), jnp.int32),
            pltpu.VMEM((LANES,), jnp.int32),
            pltpu.SemaphoreType.DMA,
            pltpu.SemaphoreType.DMA,
        ],
    )


def _tc_reduce_body(y_ref, acc_ref):
    i = pl.program_id(0)

    @pl.when(i == 0)
    def _():
        acc_ref[...] = jnp.zeros_like(acc_ref)

    b = y_ref[...].reshape(4, TC_BLOCK_ROWS // 32, 8, TC_COLS)
    acc_ref[...] += (
        (jnp.sum(b[0], axis=0) + jnp.sum(b[1], axis=0))
        + (jnp.sum(b[2], axis=0) + jnp.sum(b[3], axis=0))
    )


_tc_reduce = pl.pallas_call(
    _tc_reduce_body,
    grid=(TC_GRID,),
    in_specs=[pl.BlockSpec((TC_BLOCK_ROWS, TC_COLS), lambda i: (TC_BLOCK_OFF + i, 0))],
    out_specs=pl.BlockSpec(memory_space=pltpu.VMEM),
    out_shape=jax.ShapeDtypeStruct((8, TC_COLS), jnp.int32),
)


def _tc_finalize_body(part_ref, tcsum_ref, counts_ref, nc_ref, w_ref):
    total_pos = (jnp.sum(part_ref[...]) + jnp.sum(tcsum_ref[...])).astype(jnp.float32)
    neg = jnp.float32(N_TOTAL) - total_pos
    c0 = counts_ref[0]
    c1 = counts_ref[1]
    n0 = BETA * c0 + (1.0 - BETA) * neg
    n1 = BETA * c1 + (1.0 - BETA) * total_pos
    s0 = n0 + 1.0
    s1 = n1 + 1.0
    tot = s0 + s1
    w0 = tot / (NUM_CLASSES * s0)
    w1 = tot / (NUM_CLASSES * s1)
    wm = (w0 + w1) * 0.5
    w0n = w0 / (wm + 1e-8)
    w1n = w1 / (wm + 1e-8)
    nc_ref[0] = n0
    nc_ref[1] = n1
    w_ref[0] = w0n
    w_ref[1] = w1n


_tc_finalize = pl.pallas_call(
    _tc_finalize_body,
    out_shape=(
        jax.ShapeDtypeStruct((NUM_CLASSES,), jnp.float32),
        jax.ShapeDtypeStruct((NUM_CLASSES,), jnp.float32),
    ),
    in_specs=[
        pl.BlockSpec(memory_space=pltpu.VMEM),
        pl.BlockSpec(memory_space=pltpu.VMEM),
        pl.BlockSpec(memory_space=pltpu.SMEM),
    ],
    out_specs=(
        pl.BlockSpec(memory_space=pltpu.SMEM),
        pl.BlockSpec(memory_space=pltpu.SMEM),
    ),
)


@jax.jit
def kernel(y, counts):
    y = y.astype(jnp.int32)
    part_sc = _sc_partial_sums()(y)
    tc_sum = _tc_reduce(y.reshape(N_TOTAL // TC_COLS, TC_COLS))
    new_counts, weights = _tc_finalize(part_sc.reshape(4, 128), tc_sum, counts)
    return new_counts, weights
